# baseline (speedup 1.0000x reference)
import functools

import jax
import jax.numpy as jnp
from jax import lax
from jax.experimental import pallas as pl
from jax.experimental.pallas import tpu as pltpu
from jax.experimental.pallas import tpu_sc as plsc

_L = 16


def _sc_add(Bn, S, D, NC, NS):
    NW = NC * NS
    rows_per_w = S // NW
    R = 32
    nblk = rows_per_w // R
    mesh = plsc.VectorSubcoreMesh(core_axis_name="c", subcore_axis_name="s")

    @functools.partial(
        pl.kernel,
        mesh=mesh,
        out_type=jax.ShapeDtypeStruct((Bn, S, D), jnp.float32),
        scratch_types=[
            pltpu.VMEM((R, D), jnp.float32),
            pltpu.VMEM((R, D), jnp.float32),
        ],
    )
    def k(x_hbm, tab_hbm, out_hbm, tab_v, io_v):
        wid = lax.axis_index("s") * NC + lax.axis_index("c")
        s0 = wid * rows_per_w

        def block(i, _):
            row0 = s0 + i * R
            pltpu.sync_copy(tab_hbm.at[pl.ds(row0, R), :], tab_v)
            for b in range(Bn):
                pltpu.sync_copy(x_hbm.at[b, pl.ds(row0, R), :], io_v)

                @plsc.parallel_loop(0, R)
                def add_row(r):
                    for c in range(D // _L):
                        sl = pl.ds(c * _L, _L)
                        plsc.addupdate(io_v.at[r, sl], tab_v[r, sl])

                pltpu.sync_copy(io_v, out_hbm.at[b, pl.ds(row0, R), :])
            return 0

        lax.fori_loop(0, nblk, block, 0)

    return k


def _tc_body(x_ref, t_ref, o_ref):
    o_ref[...] = x_ref[...] + t_ref[...]


def _tc_add(Bn, S, D, blk=512):
    return pl.pallas_call(
        _tc_body,
        grid=(S // blk, Bn),
        in_specs=[
            pl.BlockSpec((1, blk, D), lambda s, b: (b, s, 0)),
            pl.BlockSpec((blk, D), lambda s, b: (s, 0)),
        ],
        out_specs=pl.BlockSpec((1, blk, D), lambda s, b: (b, s, 0)),
        out_shape=jax.ShapeDtypeStruct((Bn, S, D), jnp.float32),
    )


def kernel(inputs, pos_table):
    B, S, D = inputs.shape
    info = plsc.get_sparse_core_info()
    NC, NS = info.num_cores, info.num_subcores
    B_TC = 3
    tc_out = _tc_add(B_TC, S, D)(inputs[:B_TC], pos_table[:S])
    sc_out = _sc_add(B - B_TC, S, D, NC, NS)(inputs[B_TC:], pos_table)
    return jnp.concatenate([tc_out, sc_out], axis=0)


# speedup vs baseline: 2.1088x; 2.1088x over previous
import functools

import jax
import jax.numpy as jnp
from jax import lax
from jax.experimental import pallas as pl
from jax.experimental.pallas import tpu as pltpu
from jax.experimental.pallas import tpu_sc as plsc

_L = 16


def _sc_add(B0, B, S, D, NC, NS):
    NW = NC * NS
    rows_per_w = S // NW
    R = 32
    nblk = rows_per_w // R
    mesh = plsc.VectorSubcoreMesh(core_axis_name="c", subcore_axis_name="s")

    @functools.partial(
        pl.kernel,
        mesh=mesh,
        out_type=jax.ShapeDtypeStruct((B - B0, S, D), jnp.float32),
        scratch_types=[
            pltpu.VMEM((R, D), jnp.float32),
            pltpu.VMEM((R, D), jnp.float32),
        ],
    )
    def k(x_hbm, tab_hbm, out_hbm, tab_v, io_v):
        wid = lax.axis_index("s") * NC + lax.axis_index("c")
        s0 = wid * rows_per_w

        def block(i, _):
            row0 = s0 + i * R
            pltpu.sync_copy(tab_hbm.at[pl.ds(row0, R), :], tab_v)
            for b in range(B0, B):
                pltpu.sync_copy(x_hbm.at[b, pl.ds(row0, R), :], io_v)

                @plsc.parallel_loop(0, R)
                def add_row(r):
                    for c in range(D // _L):
                        sl = pl.ds(c * _L, _L)
                        plsc.addupdate(io_v.at[r, sl], tab_v[r, sl])

                pltpu.sync_copy(io_v, out_hbm.at[b - B0, pl.ds(row0, R), :])
            return 0

        lax.fori_loop(0, nblk, block, 0)

    return k


def _tc_body(x_ref, t_ref, o_ref):
    o_ref[...] = x_ref[...] + t_ref[...]


def _tc_add(B_TC, B, S, D, blk=512):
    return pl.pallas_call(
        _tc_body,
        grid=(S // blk, B_TC),
        in_specs=[
            pl.BlockSpec((1, blk, D), lambda s, b: (b, s, 0)),
            pl.BlockSpec((blk, D), lambda s, b: (s, 0)),
        ],
        out_specs=pl.BlockSpec((1, blk, D), lambda s, b: (b, s, 0)),
        out_shape=jax.ShapeDtypeStruct((B, S, D), jnp.float32),
    )


def _merge_body(full_ref, sc_ref, o_ref):
    o_ref[...] = sc_ref[...]


def _merge(B_TC, B, S, D, blk=512):
    return pl.pallas_call(
        _merge_body,
        grid=(B - B_TC, S // blk),
        in_specs=[
            pl.BlockSpec(memory_space=pl.ANY),
            pl.BlockSpec((1, blk, D), lambda b, s: (b, s, 0)),
        ],
        out_specs=pl.BlockSpec((1, blk, D), lambda b, s: (b + B_TC, s, 0)),
        out_shape=jax.ShapeDtypeStruct((B, S, D), jnp.float32),
        input_output_aliases={0: 0},
    )


def kernel(inputs, pos_table):
    B, S, D = inputs.shape
    info = plsc.get_sparse_core_info()
    NC, NS = info.num_cores, info.num_subcores
    B_TC = 3
    tc_out = _tc_add(B_TC, B, S, D)(inputs, pos_table)
    sc_out = _sc_add(B_TC, B, S, D, NC, NS)(inputs, pos_table)
    return _merge(B_TC, B, S, D)(tc_out, sc_out)


# speedup vs baseline: 2.2340x; 1.0594x over previous
import functools

import jax
import jax.numpy as jnp
from jax import lax
from jax.experimental import pallas as pl
from jax.experimental.pallas import tpu as pltpu
from jax.experimental.pallas import tpu_sc as plsc

_L = 16
_NIO = 5


def _sc_add(S0, B, S, D, NC, NS, R=32):
    NW = NC * NS
    Ssc = S - S0
    rows_per_w = Ssc // NW
    nblk = rows_per_w // R
    mesh = plsc.VectorSubcoreMesh(core_axis_name="c", subcore_axis_name="s")

    @functools.partial(
        pl.kernel,
        mesh=mesh,
        out_type=jax.ShapeDtypeStruct((B, Ssc, D), jnp.float32),
        scratch_types=[
            pltpu.VMEM((R, D), jnp.float32),
            pltpu.VMEM((R, D), jnp.float32),
        ],
    )
    def k(x_hbm, tab_hbm, out_hbm, tab_v, io_v):
        wid = lax.axis_index("s") * NC + lax.axis_index("c")
        w0 = wid * rows_per_w

        def block(i, _):
            row0 = w0 + i * R
            pltpu.sync_copy(tab_hbm.at[pl.ds(S0 + row0, R), :], tab_v)
            for b in range(B):
                pltpu.sync_copy(x_hbm.at[b, pl.ds(S0 + row0, R), :], io_v)

                @plsc.parallel_loop(0, R)
                def add_row(r):
                    for c in range(D // _L):
                        sl = pl.ds(c * _L, _L)
                        plsc.addupdate(io_v.at[r, sl], tab_v[r, sl])

                pltpu.sync_copy(io_v, out_hbm.at[b, pl.ds(row0, R), :])
            return 0

        lax.fori_loop(0, nblk, block, 0)

    return k


def _tc_body(x_ref, t_ref, o_ref):
    o_ref[...] = x_ref[...] + t_ref[...]


def _tc_add(S0, B, S, D, blk=512):
    return pl.pallas_call(
        _tc_body,
        grid=(S0 // blk, B),
        in_specs=[
            pl.BlockSpec((1, blk, D), lambda s, b: (b, s, 0)),
            pl.BlockSpec((blk, D), lambda s, b: (s, 0)),
        ],
        out_specs=pl.BlockSpec((1, blk, D), lambda s, b: (b, s, 0)),
        out_shape=jax.ShapeDtypeStruct((B, S, D), jnp.float32),
    )


def _merge_body(full_ref, sc_ref, o_ref):
    o_ref[...] = sc_ref[...]


def _merge(S0, B, S, D, blk=512):
    return pl.pallas_call(
        _merge_body,
        grid=(B, (S - S0) // blk),
        in_specs=[
            pl.BlockSpec(memory_space=pl.ANY),
            pl.BlockSpec((1, blk, D), lambda b, s: (b, s, 0)),
        ],
        out_specs=pl.BlockSpec((1, blk, D), lambda b, s: (b, S0 // blk + s, 0)),
        out_shape=jax.ShapeDtypeStruct((B, S, D), jnp.float32),
        input_output_aliases={0: 0},
    )


def kernel(inputs, pos_table):
    B, S, D = inputs.shape
    info = plsc.get_sparse_core_info()
    NC, NS = info.num_cores, info.num_subcores
    S0 = (3 * S) // 4
    tc_out = _tc_add(S0, B, S, D)(inputs, pos_table)
    sc_out = _sc_add(S0, B, S, D, NC, NS)(inputs, pos_table)
    return _merge(S0, B, S, D)(tc_out, sc_out)


# speedup vs baseline: 2.3655x; 1.0589x over previous
import functools

import jax
import jax.numpy as jnp
from jax import lax
from jax.experimental import pallas as pl
from jax.experimental.pallas import tpu as pltpu
from jax.experimental.pallas import tpu_sc as plsc

_L = 16
_NIO = 5


def _sc_add(S0, B, S, D, NC, NS, R=32):
    NW = NC * NS
    Ssc = S - S0
    rows_per_w = Ssc // NW
    nblk = rows_per_w // R
    mesh = plsc.VectorSubcoreMesh(core_axis_name="c", subcore_axis_name="s")

    @functools.partial(
        pl.kernel,
        mesh=mesh,
        out_type=jax.ShapeDtypeStruct((B, Ssc, D), jnp.float32),
        scratch_types=[
            pltpu.VMEM((R, D), jnp.float32),
            pltpu.VMEM((R, D), jnp.float32),
        ],
    )
    def k(x_hbm, tab_hbm, out_hbm, tab_v, io_v):
        wid = lax.axis_index("s") * NC + lax.axis_index("c")
        w0 = wid * rows_per_w

        def block(i, _):
            row0 = w0 + i * R
            pltpu.sync_copy(tab_hbm.at[pl.ds(S0 + row0, R), :], tab_v)
            for b in range(B):
                pltpu.sync_copy(x_hbm.at[b, pl.ds(S0 + row0, R), :], io_v)

                @plsc.parallel_loop(0, R)
                def add_row(r):
                    for c in range(D // _L):
                        sl = pl.ds(c * _L, _L)
                        plsc.addupdate(io_v.at[r, sl], tab_v[r, sl])

                pltpu.sync_copy(io_v, out_hbm.at[b, pl.ds(row0, R), :])
            return 0

        lax.fori_loop(0, nblk, block, 0)

    return k


def _tc_body(x_ref, t_ref, o_ref):
    o_ref[...] = x_ref[...] + t_ref[...]


def _tc_add(S0, B, S, D, blk=1024):
    return pl.pallas_call(
        _tc_body,
        grid=(S0 // blk, B),
        in_specs=[
            pl.BlockSpec((1, blk, D), lambda s, b: (b, s, 0)),
            pl.BlockSpec((blk, D), lambda s, b: (s, 0)),
        ],
        out_specs=pl.BlockSpec((1, blk, D), lambda s, b: (b, s, 0)),
        out_shape=jax.ShapeDtypeStruct((B, S, D), jnp.float32),
    )


def _merge_body(full_ref, sc_ref, o_ref):
    o_ref[...] = sc_ref[...]


def _merge(S0, B, S, D, blk=1024):
    return pl.pallas_call(
        _merge_body,
        grid=(B, (S - S0) // blk),
        in_specs=[
            pl.BlockSpec(memory_space=pl.ANY),
            pl.BlockSpec((1, blk, D), lambda b, s: (b, s, 0)),
        ],
        out_specs=pl.BlockSpec((1, blk, D), lambda b, s: (b, S0 // blk + s, 0)),
        out_shape=jax.ShapeDtypeStruct((B, S, D), jnp.float32),
        input_output_aliases={0: 0},
    )


def kernel(inputs, pos_table):
    B, S, D = inputs.shape
    info = plsc.get_sparse_core_info()
    NC, NS = info.num_cores, info.num_subcores
    S0 = (3 * S) // 4
    tc_out = _tc_add(S0, B, S, D)(inputs, pos_table)
    sc_out = _sc_add(S0, B, S, D, NC, NS)(inputs, pos_table)
    return _merge(S0, B, S, D)(tc_out, sc_out)


# speedup vs baseline: 2.4328x; 1.0284x over previous
import functools

import jax
import jax.numpy as jnp
from jax import lax
from jax.experimental import pallas as pl
from jax.experimental.pallas import tpu as pltpu
from jax.experimental.pallas import tpu_sc as plsc

_L = 16
_NIO = 5


def _sc_add(S0, B, S, D, NC, NS, R=32):
    NW = NC * NS
    Ssc = S - S0
    rows_per_w = Ssc // NW
    nblk = rows_per_w // R
    mesh = plsc.VectorSubcoreMesh(core_axis_name="c", subcore_axis_name="s")

    @functools.partial(
        pl.kernel,
        mesh=mesh,
        out_type=jax.ShapeDtypeStruct((B, Ssc, D), jnp.float32),
        scratch_types=[
            pltpu.VMEM((R, D), jnp.float32),
            pltpu.VMEM((R, D), jnp.float32),
        ],
    )
    def k(x_hbm, tab_hbm, out_hbm, tab_v, io_v):
        wid = lax.axis_index("s") * NC + lax.axis_index("c")
        w0 = wid * rows_per_w

        def block(i, _):
            row0 = w0 + i * R
            pltpu.sync_copy(tab_hbm.at[pl.ds(S0 + row0, R), :], tab_v)
            for b in range(B):
                pltpu.sync_copy(x_hbm.at[b, pl.ds(S0 + row0, R), :], io_v)

                @plsc.parallel_loop(0, R)
                def add_row(r):
                    for c in range(D // _L):
                        sl = pl.ds(c * _L, _L)
                        plsc.addupdate(io_v.at[r, sl], tab_v[r, sl])

                pltpu.sync_copy(io_v, out_hbm.at[b, pl.ds(row0, R), :])
            return 0

        lax.fori_loop(0, nblk, block, 0)

    return k


def _tc_body(x_ref, t_ref, o_ref):
    o_ref[...] = x_ref[...] + t_ref[...]


def _tc_add(S0, B, S, D, blk=2048):
    return pl.pallas_call(
        _tc_body,
        grid=(S0 // blk, B),
        in_specs=[
            pl.BlockSpec((1, blk, D), lambda s, b: (b, s, 0)),
            pl.BlockSpec((blk, D), lambda s, b: (s, 0)),
        ],
        out_specs=pl.BlockSpec((1, blk, D), lambda s, b: (b, s, 0)),
        out_shape=jax.ShapeDtypeStruct((B, S, D), jnp.float32),
    )


def _merge_body(full_ref, sc_ref, o_ref):
    o_ref[...] = sc_ref[...]


def _merge(S0, B, S, D, blk=2048):
    return pl.pallas_call(
        _merge_body,
        grid=(B, (S - S0) // blk),
        in_specs=[
            pl.BlockSpec(memory_space=pl.ANY),
            pl.BlockSpec((1, blk, D), lambda b, s: (b, s, 0)),
        ],
        out_specs=pl.BlockSpec((1, blk, D), lambda b, s: (b, S0 // blk + s, 0)),
        out_shape=jax.ShapeDtypeStruct((B, S, D), jnp.float32),
        input_output_aliases={0: 0},
    )


def kernel(inputs, pos_table):
    B, S, D = inputs.shape
    info = plsc.get_sparse_core_info()
    NC, NS = info.num_cores, info.num_subcores
    S0 = (3 * S) // 4
    tc_out = _tc_add(S0, B, S, D)(inputs, pos_table)
    sc_out = _sc_add(S0, B, S, D, NC, NS)(inputs, pos_table)
    return _merge(S0, B, S, D)(tc_out, sc_out)
